# Initial kernel scaffold; baseline (speedup 1.0000x reference)
import functools

import jax
import jax.numpy as jnp
from jax.experimental import pallas as pl
from jax.experimental.pallas import tpu as pltpu

NZ, NY, NX = 16, 256, 256
W_CONC = 1.0
W_WIND = 50.0
W_PCC = 1.0
W_PHYS = 0.1
TOPK_RATIO = 0.1
CONC_WEIGHT_SCALE = 10.0
DX, DY = 100.0, 100.0


def _k1_body(p_ref, t_ref, w_ref, part_ref):
    p = p_ref[0]
    t = t_ref[0]
    d = p - t
    pix = d * d
    sp = jnp.log(1.0 + jnp.exp(-jnp.abs(t))) + jnp.maximum(t, 0.0)
    vw = 1.0 + CONC_WEIGHT_SCALE * sp
    aw = jnp.where(t > p, 3.0, 1.0)
    w = pix * vw * aw
    w_ref[0] = w
    bits = jax.lax.bitcast_convert_type(w, jnp.int32)
    scalars = [
        jnp.sum(p),
        jnp.sum(t),
        jnp.sum(p * t),
        jnp.sum(p * p),
        jnp.sum(t * t),
    ]
    for j in range(16):
        thr = j << 27
        scalars.append(jnp.sum((bits >= thr).astype(jnp.float32)))
    lane = jax.lax.broadcasted_iota(jnp.int32, (1, 1, 128), 2)
    vec = jnp.zeros((1, 1, 128), jnp.float32)
    for m, val in enumerate(scalars):
        vec = vec + jnp.where(lane == m, val, 0.0)
    part_ref[...] = vec


def _k2_body(pw_ref, tw_ref, c_ref, part_ref):
    pw = pw_ref[0]
    tw = tw_ref[0]
    c = c_ref[0]
    dw = pw - tw
    acc_w = jnp.sum(dw * dw)
    row = jax.lax.broadcasted_iota(jnp.int32, (NX, 3 * NX), 0)
    lane3 = jax.lax.broadcasted_iota(jnp.int32, (NX, 3 * NX), 1)
    sel = (lane3 // 3 == row).astype(jnp.float32)
    c3 = jax.lax.dot(c, sel, precision=jax.lax.Precision.HIGHEST)
    cr = jnp.concatenate([c3[:, 3:], c3[:, 3 * NX - 3:]], axis=1)
    cl = jnp.concatenate([c3[:, :3], c3[:, : 3 * NX - 3]], axis=1)
    dcdx3 = (cr - cl) * (1.0 / (2.0 * DX))
    cu = jnp.concatenate([c3[1:, :], c3[NY - 1:, :]], axis=0)
    cd = jnp.concatenate([c3[:1, :], c3[: NY - 1, :]], axis=0)
    dcdy3 = (cu - cd) * (1.0 / (2.0 * DY))
    lmod = jax.lax.broadcasted_iota(jnp.int32, (NY, 3 * NX), 1) % 3
    dsel = jnp.where(lmod == 0, dcdx3, jnp.where(lmod == 1, dcdy3, 0.0))
    prod = pw * dsel
    s1 = jnp.sum(prod * prod)
    pshift = jnp.concatenate(
        [prod[:, 1:], jnp.zeros((NY, 1), jnp.float32)], axis=1)
    cross = jnp.sum(prod * pshift)
    acc_p = s1 + 2.0 * cross
    lane = jax.lax.broadcasted_iota(jnp.int32, (1, 1, 128), 2)
    vec = jnp.where(lane == 0, acc_w, 0.0) + jnp.where(lane == 1, acc_p, 0.0)
    part_ref[...] = vec


def _k3_body(params_ref, w_ref, part_ref, *, with_sums):
    lo = params_ref[0]
    step = params_ref[1]
    w = w_ref[0]
    bits = jax.lax.bitcast_convert_type(w, jnp.int32)
    scalars = []
    for j in range(16):
        thr = lo + j * step
        scalars.append(jnp.sum((bits >= thr).astype(jnp.float32)))
    if with_sums:
        for j in range(16):
            thr = lo + j * step
            scalars.append(jnp.sum(jnp.where(bits >= thr, w, 0.0)))
    lane = jax.lax.broadcasted_iota(jnp.int32, (1, 1, 128), 2)
    vec = jnp.zeros((1, 1, 128), jnp.float32)
    for m, val in enumerate(scalars):
        vec = vec + jnp.where(lane == m, val, 0.0)
    part_ref[...] = vec


def _count_pass(w_r, lo, step, with_sums):
    params = jnp.stack([lo, step]).astype(jnp.int32)
    grid_spec = pltpu.PrefetchScalarGridSpec(
        num_scalar_prefetch=1,
        grid=(32,),
        in_specs=[pl.BlockSpec((1, 128, 1024), lambda i, params: (i, 0, 0))],
        out_specs=pl.BlockSpec((1, 1, 128), lambda i, params: (i, 0, 0)),
    )
    part = pl.pallas_call(
        functools.partial(_k3_body, with_sums=with_sums),
        grid_spec=grid_spec,
        out_shape=jax.ShapeDtypeStruct((32, 1, 128), jnp.float32),
    )(params, w_r)
    return jnp.sum(part[:, 0, :], axis=0)


def _refine(lo, step, cnts, k):
    idx = jnp.arange(16, dtype=jnp.int32)
    jstar = jnp.max(jnp.where(cnts >= k, idx, 0))
    return lo + jstar * step


def kernel(pred_wind, true_wind, pred_conc, true_conc):
    B = pred_conc.shape[0]
    n_conc = pred_conc.shape[1]
    n_wind = pred_wind.shape[1]
    total_conc = B * n_conc
    k = max(1, int(total_conc * TOPK_RATIO))

    pc = pred_conc.reshape(B, 1024, 1024)
    tc = true_conc.reshape(B, 1024, 1024)

    w, part1 = pl.pallas_call(
        _k1_body,
        grid=(B, 8),
        in_specs=[
            pl.BlockSpec((1, 128, 1024), lambda i, j: (i, j, 0)),
            pl.BlockSpec((1, 128, 1024), lambda i, j: (i, j, 0)),
        ],
        out_specs=[
            pl.BlockSpec((1, 128, 1024), lambda i, j: (i, j, 0)),
            pl.BlockSpec((1, 1, 128), lambda i, j: (i * 8 + j, 0, 0)),
        ],
        out_shape=[
            jax.ShapeDtypeStruct((B, 1024, 1024), jnp.float32),
            jax.ShapeDtypeStruct((B * 8, 1, 128), jnp.float32),
        ],
    )(pc, tc)

    pw = pred_wind.reshape(B * NZ, NY, 3 * NX)
    tw = true_wind.reshape(B * NZ, NY, 3 * NX)
    cz = pred_conc.reshape(B * NZ, NY, NX)
    part2 = pl.pallas_call(
        _k2_body,
        grid=(B * NZ,),
        in_specs=[
            pl.BlockSpec((1, NY, 3 * NX), lambda i: (i, 0, 0)),
            pl.BlockSpec((1, NY, 3 * NX), lambda i: (i, 0, 0)),
            pl.BlockSpec((1, NY, NX), lambda i: (i, 0, 0)),
        ],
        out_specs=pl.BlockSpec((1, 1, 128), lambda i: (i, 0, 0)),
        out_shape=jax.ShapeDtypeStruct((B * NZ, 1, 128), jnp.float32),
    )(pw, tw, cz)

    p1 = part1[:, 0, :].reshape(B, 8, 128).sum(axis=1)
    n = jnp.float32(n_conc)
    s_p, s_t, s_pt, s_pp, s_tt = (p1[:, m] for m in range(5))
    num = s_pt - s_p * s_t / n
    var_p = jnp.maximum(s_pp - s_p * s_p / n, 0.0)
    var_t = jnp.maximum(s_tt - s_t * s_t / n, 0.0)
    den = jnp.sqrt(var_p) * jnp.sqrt(var_t) + 1e-08
    loss_pcc = 1.0 - jnp.mean(num / den)

    p2 = part2[:, 0, :].sum(axis=0)
    loss_w = p2[0] / jnp.float32(B * n_wind)
    loss_phys = p2[1] / jnp.float32(B * NZ * NY * NX)

    cnts0 = part1[:, 0, 5:21].sum(axis=0)
    lo = _refine(jnp.int32(0), jnp.int32(1 << 27), cnts0, k)
    step = jnp.int32(1 << 23)
    w_r = w.reshape(32, 128, 1024)
    for _ in range(4):
        cnts = _count_pass(w_r, lo, step, with_sums=False)
        lo = _refine(lo, step, cnts, k)
        step = step // 16
    final = _count_pass(w_r, lo, step, with_sums=True)
    cnts, sums = final[:16], final[16:32]
    idx = jnp.arange(16, dtype=jnp.int32)
    jstar = jnp.max(jnp.where(cnts >= k, idx, 0))
    cstar = jnp.sum(jnp.where(idx == jstar, cnts, 0.0))
    sstar = jnp.sum(jnp.where(idx == jstar, sums, 0.0))
    t_bits = lo + jstar * step + step // 2
    t_mid = jax.lax.bitcast_convert_type(t_bits, jnp.float32)
    sum_topk = sstar - (cstar - jnp.float32(k)) * t_mid
    loss_c = sum_topk / jnp.float32(k)

    total = (W_CONC * loss_c + W_WIND * loss_w + W_PCC * loss_pcc
             + W_PHYS * loss_phys)
    return (total, loss_c, loss_w, loss_pcc, loss_phys)


# speedup vs baseline: 10.0400x; 10.0400x over previous
import functools

import jax
import jax.numpy as jnp
from jax.experimental import pallas as pl
from jax.experimental.pallas import tpu as pltpu

NZ, NY, NX = 16, 256, 256
W_CONC = 1.0
W_WIND = 50.0
W_PCC = 1.0
W_PHYS = 0.1
TOPK_RATIO = 0.1
CONC_WEIGHT_SCALE = 10.0
DX, DY = 100.0, 100.0


def _k1_body(p_ref, t_ref, w_ref, part_ref):
    p = p_ref[0]
    t = t_ref[0]
    d = p - t
    pix = d * d
    sp = jnp.log(1.0 + jnp.exp(-jnp.abs(t))) + jnp.maximum(t, 0.0)
    vw = 1.0 + CONC_WEIGHT_SCALE * sp
    aw = jnp.where(t > p, 3.0, 1.0)
    w = pix * vw * aw
    w_ref[0] = w
    bits = jax.lax.bitcast_convert_type(w, jnp.int32)
    scalars = [
        jnp.sum(p),
        jnp.sum(t),
        jnp.sum(p * t),
        jnp.sum(p * p),
        jnp.sum(t * t),
    ]
    for j in range(16):
        thr = j << 27
        scalars.append(jnp.sum((bits >= thr).astype(jnp.float32)))
    lane = jax.lax.broadcasted_iota(jnp.int32, (1, 1, 128), 2)
    vec = jnp.zeros((1, 1, 128), jnp.float32)
    for m, val in enumerate(scalars):
        vec = vec + jnp.where(lane == m, val, 0.0)
    part_ref[...] = vec


def _k2_body(pw_ref, tw_ref, c_ref, part_ref):
    pw = pw_ref[0]
    tw = tw_ref[0]
    c = c_ref[0]
    dw = pw - tw
    acc_w = jnp.sum(dw * dw)
    row = jax.lax.broadcasted_iota(jnp.int32, (NX, 3 * NX), 0)
    lane3 = jax.lax.broadcasted_iota(jnp.int32, (NX, 3 * NX), 1)
    sel = (lane3 // 3 == row).astype(jnp.float32)
    c3 = jax.lax.dot(c, sel, precision=jax.lax.Precision.HIGHEST)
    cr = jnp.concatenate([c3[:, 3:], c3[:, 3 * NX - 3:]], axis=1)
    cl = jnp.concatenate([c3[:, :3], c3[:, : 3 * NX - 3]], axis=1)
    dcdx3 = (cr - cl) * (1.0 / (2.0 * DX))
    cu = jnp.concatenate([c3[1:, :], c3[NY - 1:, :]], axis=0)
    cd = jnp.concatenate([c3[:1, :], c3[: NY - 1, :]], axis=0)
    dcdy3 = (cu - cd) * (1.0 / (2.0 * DY))
    lmod = jax.lax.broadcasted_iota(jnp.int32, (NY, 3 * NX), 1) % 3
    dsel = jnp.where(lmod == 0, dcdx3, jnp.where(lmod == 1, dcdy3, 0.0))
    prod = pw * dsel
    s1 = jnp.sum(prod * prod)
    pshift = jnp.concatenate(
        [prod[:, 1:], jnp.zeros((NY, 1), jnp.float32)], axis=1)
    cross = jnp.sum(prod * pshift)
    acc_p = s1 + 2.0 * cross
    lane = jax.lax.broadcasted_iota(jnp.int32, (1, 1, 128), 2)
    vec = jnp.where(lane == 0, acc_w, 0.0) + jnp.where(lane == 1, acc_p, 0.0)
    part_ref[...] = vec


def _k3_body(params_ref, w_ref, part_ref, *, with_sums):
    lo = params_ref[0]
    step = params_ref[1]
    w = w_ref[0]
    bits = jax.lax.bitcast_convert_type(w, jnp.int32)
    scalars = []
    for j in range(16):
        thr = lo + j * step
        scalars.append(jnp.sum((bits >= thr).astype(jnp.float32)))
    if with_sums:
        for j in range(16):
            thr = lo + j * step
            scalars.append(jnp.sum(jnp.where(bits >= thr, w, 0.0)))
    lane = jax.lax.broadcasted_iota(jnp.int32, (1, 1, 128), 2)
    vec = jnp.zeros((1, 1, 128), jnp.float32)
    for m, val in enumerate(scalars):
        vec = vec + jnp.where(lane == m, val, 0.0)
    part_ref[...] = vec


def _count_pass(w_r, lo, step, with_sums):
    params = jnp.stack([lo, step]).astype(jnp.int32)
    grid_spec = pltpu.PrefetchScalarGridSpec(
        num_scalar_prefetch=1,
        grid=(32,),
        in_specs=[pl.BlockSpec((1, 128, 1024), lambda i, params: (i, 0, 0))],
        out_specs=pl.BlockSpec((1, 1, 128), lambda i, params: (i, 0, 0)),
    )
    part = pl.pallas_call(
        functools.partial(_k3_body, with_sums=with_sums),
        grid_spec=grid_spec,
        out_shape=jax.ShapeDtypeStruct((32, 1, 128), jnp.float32),
    )(params, w_r)
    return jnp.sum(part[:, 0, :], axis=0)


def _refine(lo, step, cnts, k):
    idx = jnp.arange(16, dtype=jnp.int32)
    jstar = jnp.max(jnp.where(cnts >= k, idx, 0))
    return lo + jstar * step


def kernel(pred_wind, true_wind, pred_conc, true_conc):
    B = pred_conc.shape[0]
    n_conc = pred_conc.shape[1]
    n_wind = pred_wind.shape[1]
    total_conc = B * n_conc
    k = max(1, int(total_conc * TOPK_RATIO))

    pc = pred_conc.reshape(B, 1024, 1024)
    tc = true_conc.reshape(B, 1024, 1024)

    w, part1 = pl.pallas_call(
        _k1_body,
        grid=(B, 8),
        in_specs=[
            pl.BlockSpec((1, 128, 1024), lambda i, j: (i, j, 0)),
            pl.BlockSpec((1, 128, 1024), lambda i, j: (i, j, 0)),
        ],
        out_specs=[
            pl.BlockSpec((1, 128, 1024), lambda i, j: (i, j, 0)),
            pl.BlockSpec((1, 1, 128), lambda i, j: (i * 8 + j, 0, 0)),
        ],
        out_shape=[
            jax.ShapeDtypeStruct((B, 1024, 1024), jnp.float32),
            jax.ShapeDtypeStruct((B * 8, 1, 128), jnp.float32),
        ],
    )(pc, tc)

    pw = pred_wind.reshape(B * NZ, NY, 3 * NX)
    tw = true_wind.reshape(B * NZ, NY, 3 * NX)
    cz = pred_conc.reshape(B * NZ, NY, NX)
    part2 = pl.pallas_call(
        _k2_body,
        grid=(B * NZ,),
        in_specs=[
            pl.BlockSpec((1, NY, 3 * NX), lambda i: (i, 0, 0)),
            pl.BlockSpec((1, NY, 3 * NX), lambda i: (i, 0, 0)),
            pl.BlockSpec((1, NY, NX), lambda i: (i, 0, 0)),
        ],
        out_specs=pl.BlockSpec((1, 1, 128), lambda i: (i, 0, 0)),
        out_shape=jax.ShapeDtypeStruct((B * NZ, 1, 128), jnp.float32),
    )(pw, tw, cz)

    p1 = part1[:, 0, :].reshape(B, 8, 128).sum(axis=1)
    n = jnp.float32(n_conc)
    s_p, s_t, s_pt, s_pp, s_tt = (p1[:, m] for m in range(5))
    num = s_pt - s_p * s_t / n
    var_p = jnp.maximum(s_pp - s_p * s_p / n, 0.0)
    var_t = jnp.maximum(s_tt - s_t * s_t / n, 0.0)
    den = jnp.sqrt(var_p) * jnp.sqrt(var_t) + 1e-08
    loss_pcc = 1.0 - jnp.mean(num / den)

    p2 = part2[:, 0, :].sum(axis=0)
    loss_w = p2[0] / jnp.float32(B * n_wind)
    loss_phys = p2[1] / jnp.float32(B * NZ * NY * NX)

    cnts0 = part1[:, 0, 5:21].sum(axis=0)
    lo = _refine(jnp.int32(0), jnp.int32(1 << 27), cnts0, k)
    step = jnp.int32(1 << 23)
    w_r = w.reshape(32, 128, 1024)
    for _ in range(4):
        cnts = _count_pass(w_r, lo, step, with_sums=False)[:16]
        lo = _refine(lo, step, cnts, k)
        step = step // 16
    final = _count_pass(w_r, lo, step, with_sums=True)
    cnts, sums = final[:16], final[16:32]
    idx = jnp.arange(16, dtype=jnp.int32)
    jstar = jnp.max(jnp.where(cnts >= k, idx, 0))
    cstar = jnp.sum(jnp.where(idx == jstar, cnts, 0.0))
    sstar = jnp.sum(jnp.where(idx == jstar, sums, 0.0))
    t_bits = lo + jstar * step + step // 2
    t_mid = jax.lax.bitcast_convert_type(t_bits, jnp.float32)
    sum_topk = sstar - (cstar - jnp.float32(k)) * t_mid
    loss_c = sum_topk / jnp.float32(k)

    total = (W_CONC * loss_c + W_WIND * loss_w + W_PCC * loss_pcc
             + W_PHYS * loss_phys)
    return (total, loss_c, loss_w, loss_pcc, loss_phys)
